# baseline (speedup 1.0000x reference)
import functools
import math

import jax
import jax.numpy as jnp
from jax import lax
from jax.experimental import pallas as pl
from jax.experimental.pallas import tpu as pltpu
from jax.experimental.pallas import tpu_sc as plsc

_D = 768
_L = 8192
_B = 4
_SCALE = math.sqrt(_D)

_NC = 2
_NS = 16
_LANES = 16
_NW = _NC * _NS
_ROWS_W = _L // _NW
_CH = 32
_NCH = _ROWS_W // _CH
_CHE = _CH * _D
_VPC = _CHE // _LANES
_UNROLL = 8
_STEPS = _NCH * _B


@functools.partial(
    pl.kernel,
    mesh=plsc.VectorSubcoreMesh(core_axis_name="c", subcore_axis_name="s"),
    out_type=jax.ShapeDtypeStruct((_B * _L * _D,), jnp.float32),
    scratch_types=[
        pltpu.VMEM((2, _CHE), jnp.float32),
        pltpu.VMEM((2, _CHE), jnp.float32),
        pltpu.VMEM((1, _CHE), jnp.float32),
        pltpu.SemaphoreType.DMA,
        pltpu.SemaphoreType.DMA,
        pltpu.SemaphoreType.DMA,
        pltpu.SemaphoreType.DMA,
        pltpu.SemaphoreType.DMA,
    ],
)
def _sc_pe_add(x_hbm, pe_hbm, out_hbm, xbuf, obuf, pebuf,
               xsem0, xsem1, osem0, osem1, pesem0):
    xsem = (xsem0, xsem1)
    osem = (osem0, osem1)
    pesem = (pesem0,)
    wid = lax.axis_index("s") * _NC + lax.axis_index("c")
    base_off = wid * (_ROWS_W * _D)

    def x_off(t):
        ch, b = divmod(t, _B)
        return b * (_L * _D) + base_off + ch * _CHE

    def start_x(t):
        s = t % 2
        return pltpu.async_copy(
            x_hbm.at[pl.ds(x_off(t), _CHE)], xbuf.at[s], xsem[s])

    def start_pe(ch):
        return pltpu.async_copy(
            pe_hbm.at[pl.ds(base_off + ch * _CHE, _CHE)], pebuf.at[0], pesem[0])

    pending_x = {0: start_x(0), 1: start_x(1)}
    pending_pe = {0: start_pe(0)}
    pending_o = {}

    for t in range(_STEPS):
        s = t % 2
        ch, b = divmod(t, _B)
        pending_x.pop(t).wait()
        if b == 0 and ch in pending_pe:
            pending_pe.pop(ch).wait()
        if t - 2 in pending_o:
            pending_o.pop(t - 2).wait()

        p = 0

        @plsc.parallel_loop(0, _CHE, step=_LANES, unroll=_UNROLL)
        def body(i):
            sl = pl.ds(i, _LANES)
            obuf[s, sl] = xbuf[s, sl] + pebuf[p, sl] * _SCALE

        pending_o[t] = pltpu.async_copy(
            obuf.at[s], out_hbm.at[pl.ds(x_off(t), _CHE)], osem[s])
        if t + 2 < _STEPS:
            pending_x[t + 2] = start_x(t + 2)
        if b == _B - 1 and ch + 1 < _NCH:
            pending_pe[ch + 1] = start_pe(ch + 1)

    for t in sorted(pending_o):
        pending_o.pop(t).wait()


def kernel(x, pe_table):
    out = _sc_pe_add(x.reshape(-1), pe_table.reshape(-1))
    return out.reshape(_B, _L, _D)


# speedup vs baseline: 1.0122x; 1.0122x over previous
import functools
import math

import jax
import jax.numpy as jnp
from jax import lax
from jax.experimental import pallas as pl
from jax.experimental.pallas import tpu as pltpu
from jax.experimental.pallas import tpu_sc as plsc

_D = 768
_L = 8192
_B = 4
_SCALE = math.sqrt(_D)

_NC = 2
_NS = 16
_LANES = 16
_NW = _NC * _NS
_ROWS_W = _L // _NW
_CH = 16
_NCH = _ROWS_W // _CH
_CHE = _CH * _D
_UNROLL = 8
_STEPS = _NCH * _B
_DEPTH = 4


@functools.partial(
    pl.kernel,
    mesh=plsc.VectorSubcoreMesh(core_axis_name="c", subcore_axis_name="s"),
    out_type=jax.ShapeDtypeStruct((_B * _L * _D,), jnp.float32),
    scratch_types=[
        pltpu.VMEM((_DEPTH, _CHE), jnp.float32),
        pltpu.VMEM((_DEPTH, _CHE), jnp.float32),
        pltpu.VMEM((2, _CHE), jnp.float32),
        pltpu.SemaphoreType.DMA,
        pltpu.SemaphoreType.DMA,
        pltpu.SemaphoreType.DMA,
        pltpu.SemaphoreType.DMA,
        pltpu.SemaphoreType.DMA,
        pltpu.SemaphoreType.DMA,
        pltpu.SemaphoreType.DMA,
        pltpu.SemaphoreType.DMA,
        pltpu.SemaphoreType.DMA,
        pltpu.SemaphoreType.DMA,
    ],
)
def _sc_pe_add(x_hbm, pe_hbm, out_hbm, xbuf, obuf, pebuf,
               xsem0, xsem1, xsem2, xsem3,
               osem0, osem1, osem2, osem3, pesem0, pesem1):
    xsem = (xsem0, xsem1, xsem2, xsem3)
    osem = (osem0, osem1, osem2, osem3)
    pesem = (pesem0, pesem1)
    wid = lax.axis_index("s") * _NC + lax.axis_index("c")
    base_off = wid * (_ROWS_W * _D)

    def x_off(t):
        ch, b = divmod(t, _B)
        return b * (_L * _D) + base_off + ch * _CHE

    def start_x(t):
        s = t % _DEPTH
        return pltpu.async_copy(
            x_hbm.at[pl.ds(x_off(t), _CHE)], xbuf.at[s], xsem[s])

    def start_pe(ch):
        p = ch % 2
        return pltpu.async_copy(
            pe_hbm.at[pl.ds(base_off + ch * _CHE, _CHE)], pebuf.at[p], pesem[p])

    pending_x = {t: start_x(t) for t in range(_DEPTH)}
    pending_pe = {0: start_pe(0), 1: start_pe(1)}
    pending_o = {}

    for t in range(_STEPS):
        s = t % _DEPTH
        ch, b = divmod(t, _B)
        pending_x.pop(t).wait()
        if b == 0 and ch in pending_pe:
            pending_pe.pop(ch).wait()
        if t - _DEPTH in pending_o:
            pending_o.pop(t - _DEPTH).wait()

        p = ch % 2

        @plsc.parallel_loop(0, _CHE, step=_LANES, unroll=_UNROLL)
        def body(i):
            sl = pl.ds(i, _LANES)
            obuf[s, sl] = xbuf[s, sl] + pebuf[p, sl] * _SCALE

        pending_o[t] = pltpu.async_copy(
            obuf.at[s], out_hbm.at[pl.ds(x_off(t), _CHE)], osem[s])
        if t + _DEPTH < _STEPS:
            pending_x[t + _DEPTH] = start_x(t + _DEPTH)
        if b == _B - 1 and ch + 2 < _NCH:
            pending_pe[ch + 2] = start_pe(ch + 2)

    for t in sorted(pending_o):
        pending_o.pop(t).wait()


def kernel(x, pe_table):
    out = _sc_pe_add(x.reshape(-1), pe_table.reshape(-1))
    return out.reshape(_B, _L, _D)


# speedup vs baseline: 5.2767x; 5.2131x over previous
import math

import jax
import jax.numpy as jnp
from jax.experimental import pallas as pl
from jax.experimental.pallas import tpu as pltpu

_D = 768
_L = 8192
_B = 4
_BL = 2048
_SCALE = math.sqrt(_D)


def _pe_add_body(x_ref, pe_ref, o_ref):
    o_ref[...] = x_ref[...] + pe_ref[...] * _SCALE


def kernel(x, pe_table):
    grid = (_L // _BL, _B)
    return pl.pallas_call(
        _pe_add_body,
        grid=grid,
        in_specs=[
            pl.BlockSpec((1, _BL, _D), lambda l, b: (b, l, 0)),
            pl.BlockSpec((_BL, _D), lambda l, b: (l, 0)),
        ],
        out_specs=pl.BlockSpec((1, _BL, _D), lambda l, b: (b, l, 0)),
        out_shape=jax.ShapeDtypeStruct((_B, _L, _D), jnp.float32),
        compiler_params=pltpu.CompilerParams(
            dimension_semantics=("arbitrary", "arbitrary"),
        ),
    )(x, pe_table)
